# baseline (speedup 1.0000x reference)
import functools

import jax
import jax.numpy as jnp
from jax import lax
from jax.experimental import pallas as pl
from jax.experimental.pallas import tpu as pltpu
from jax.experimental.pallas import tpu_sc as plsc

NC = 2
NS = 16
NW = NC * NS

LANES = 1024


def _sc_mesh():
  return plsc.VectorSubcoreMesh(
      core_axis_name="c", subcore_axis_name="s", num_cores=NC, num_subcores=NS)


_SC_PARAMS = pltpu.CompilerParams(use_tc_tiling_on_sc=False)


def _make_gather(n_rows, d, chunk):
  per_w = n_rows // NW
  assert per_w % chunk == 0
  n_ch = per_w // chunk

  @functools.partial(
      pl.kernel,
      mesh=_sc_mesh(),
      compiler_params=_SC_PARAMS,
      out_type=jax.ShapeDtypeStruct((n_rows, d), jnp.float32),
      scratch_types=[
          pltpu.VMEM((chunk,), jnp.int32),
          pltpu.VMEM((chunk, d), jnp.float32),
          pltpu.SemaphoreType.DMA,
      ],
  )
  def gather_kernel(table_hbm, idx_hbm, out_hbm, idx_v, rows_v, sem):
    wid = lax.axis_index("s") * NC + lax.axis_index("c")
    base = wid * per_w
    for j in range(n_ch):
      off = base + j * chunk
      pltpu.sync_copy(idx_hbm.at[pl.ds(off, chunk)], idx_v)
      pltpu.async_copy(table_hbm.at[idx_v], rows_v, sem).wait()
      pltpu.sync_copy(rows_v, out_hbm.at[pl.ds(off, chunk)])

  return gather_kernel


def _make_scatter_add(n_nodes, n_rows, d, chunk):
  per_w = n_rows // NW
  assert per_w % chunk == 0
  n_ch = per_w // chunk

  @functools.partial(
      pl.kernel,
      mesh=_sc_mesh(),
      compiler_params=_SC_PARAMS,
      out_type=jax.ShapeDtypeStruct((NC, n_nodes, d), jnp.float32),
      scratch_types=[
          pltpu.VMEM((chunk,), jnp.int32),
          pltpu.VMEM((chunk, d), jnp.float32),
          pltpu.VMEM_SHARED((n_nodes + 16, d), jnp.float32),
      ],
  )
  def scatter_kernel(rows_hbm, idx_hbm, zeros_hbm, out_hbm, idx_v, rows_v,
                     acc_sh):
    cid = lax.axis_index("c")
    sid = lax.axis_index("s")

    @pl.when(sid == 0)
    def _init():
      pltpu.sync_copy(zeros_hbm, acc_sh)

    plsc.subcore_barrier()
    wid = sid * NC + cid
    base = wid * per_w
    for j in range(n_ch):
      off = base + j * chunk
      pltpu.sync_copy(idx_hbm.at[pl.ds(off, chunk)], idx_v)
      pltpu.sync_copy(rows_hbm.at[pl.ds(off, chunk)], rows_v)
      pltpu.sync_copy(rows_v, acc_sh.at[idx_v], add=True)
    plsc.subcore_barrier()

    @pl.when(sid == 0)
    def _flush():
      pltpu.sync_copy(acc_sh.at[pl.ds(0, n_nodes)], out_hbm.at[cid])

  return scatter_kernel


def _make_degree(n_nodes, n_rows, chunk):
  per_w = n_rows // NW
  n_ch = per_w // chunk

  @functools.partial(
      pl.kernel,
      mesh=_sc_mesh(),
      compiler_params=_SC_PARAMS,
      out_type=jax.ShapeDtypeStruct((NC, n_nodes, 32), jnp.float32),
      scratch_types=[
          pltpu.VMEM((chunk,), jnp.int32),
          pltpu.VMEM((chunk, 32), jnp.float32),
          pltpu.VMEM_SHARED((n_nodes + 16, 32), jnp.float32),
      ],
  )
  def degree_kernel(idx_hbm, ones_hbm, zeros_hbm, out_hbm, idx_v, ones_v,
                    acc_sh):
    cid = lax.axis_index("c")
    sid = lax.axis_index("s")

    @pl.when(sid == 0)
    def _init():
      pltpu.sync_copy(zeros_hbm, acc_sh)

    pltpu.sync_copy(ones_hbm, ones_v)
    plsc.subcore_barrier()
    wid = sid * NC + cid
    base = wid * per_w
    for j in range(n_ch):
      off = base + j * chunk
      pltpu.sync_copy(idx_hbm.at[pl.ds(off, chunk)], idx_v)
      pltpu.sync_copy(ones_v, acc_sh.at[idx_v], add=True)
    plsc.subcore_barrier()

    @pl.when(sid == 0)
    def _flush():
      pltpu.sync_copy(acc_sh.at[pl.ds(0, n_nodes)], out_hbm.at[cid])

  return degree_kernel


def _embed_packed(x, W, b):
  n, din = x.shape
  d = W.shape[1]
  x4 = x.reshape(n // 4, 4 * din)
  bd = jnp.kron(jnp.eye(4, dtype=jnp.float32), W)
  b4 = jnp.tile(b, 4).reshape(1, 4 * d)

  def body(x_ref, w_ref, b_ref, o_ref):
    o_ref[...] = (
        jnp.dot(x_ref[...], w_ref[...], preferred_element_type=jnp.float32)
        + b_ref[...])

  return pl.pallas_call(
      body, out_shape=jax.ShapeDtypeStruct((n // 4, 4 * d), jnp.float32))(
          x4, bd, b4)


def _edge_messages(eaP8, xjP, K1, Kb1, K2, Kb2, K3, Kb3, d):
  rows8 = eaP8.shape[0]
  de = 16
  kw = K1.shape[1]
  rows = xjP.shape[0]
  assert rows % LANES == 0 and rows8 * 2 == rows

  k1t = K1.T
  k2t = K2.T
  k3t = K3.T.astype(jnp.bfloat16)
  kb3t = Kb3.reshape(d, d).T
  kb1c = Kb1.reshape(kw, 1)
  kb2c = Kb2.reshape(kw, 1)
  half = LANES // 2
  p1 = jnp.kron(jnp.eye(half, dtype=jnp.float32),
                jnp.array([[1.0, 0.0]], jnp.float32))
  p2 = jnp.kron(jnp.eye(half, dtype=jnp.float32),
                jnp.array([[0.0, 1.0]], jnp.float32))

  def body(ea_ref, xj_ref, k1_ref, kb1_ref, k2_ref, kb2_ref, k3_ref,
           kb3_ref, p1_ref, p2_ref, o_ref):
    eaP = jnp.transpose(ea_ref[...])
    eaQ = (
        jnp.dot(eaP[0:4 * de, :], p1_ref[...],
                preferred_element_type=jnp.float32)
        + jnp.dot(eaP[4 * de:8 * de, :], p2_ref[...],
                  preferred_element_type=jnp.float32))
    xjT = jnp.transpose(xj_ref[...])
    accs = []
    for k in range(4):
      a = jnp.dot(
          k1_ref[...], eaQ[k * de:(k + 1) * de, :],
          preferred_element_type=jnp.float32)
      a = jnp.maximum(a + kb1_ref[...], 0.0)
      a = jnp.dot(k2_ref[...], a, preferred_element_type=jnp.float32)
      a = jnp.maximum(a + kb2_ref[...], 0.0)
      wT = jnp.dot(
          k3_ref[...], a.astype(jnp.bfloat16),
          preferred_element_type=jnp.float32).astype(jnp.bfloat16)
      acc = jnp.dot(
          kb3_ref[...], xjT[k * d:(k + 1) * d, :],
          preferred_element_type=jnp.float32)
      for i in range(d):
        acc = acc + (xjT[k * d + i:k * d + i + 1, :] *
                     wT[i * d:(i + 1) * d, :].astype(jnp.float32))
      accs.append(acc)
    o_ref[...] = jnp.transpose(jnp.concatenate(accs, axis=0))

  full = lambda shape: pl.BlockSpec(shape, lambda i: tuple(0 for _ in shape))
  return pl.pallas_call(
      body,
      grid=(rows // LANES,),
      in_specs=[
          pl.BlockSpec((half, 128), lambda i: (i, 0)),
          pl.BlockSpec((LANES, 128), lambda i: (i, 0)),
          full((kw, de)),
          full((kw, 1)),
          full((kw, kw)),
          full((kw, 1)),
          full((d * d, kw)),
          full((d, d)),
          full((half, LANES)),
          full((half, LANES)),
      ],
      out_specs=pl.BlockSpec((LANES, 128), lambda i: (i, 0)),
      out_shape=jax.ShapeDtypeStruct((rows, 128), jnp.float32),
  )(eaP8, xjP, k1t, kb1c, k2t, kb2c, k3t, kb3t, p1, p2)


def _update_packed(agg2p, deg2p, hp, Wr, b, Wfin=None, bfin=None):
  rows, dl = hp.shape
  bdr = jnp.kron(jnp.eye(4, dtype=jnp.float32), Wr)
  b4 = jnp.tile(b, 4).reshape(1, dl)

  if Wfin is None:
    def body(a_ref, d_ref, h_ref, w_ref, b_ref, o_ref):
      agg = a_ref[0] + a_ref[1]
      rdeg = 1.0 / jnp.maximum(d_ref[0] + d_ref[1], 1.0)
      o_ref[...] = jnp.maximum(
          agg * rdeg
          + jnp.dot(h_ref[...], w_ref[...], preferred_element_type=jnp.float32)
          + b_ref[...], 0.0)

    return pl.pallas_call(
        body, out_shape=jax.ShapeDtypeStruct((rows, dl), jnp.float32))(
            agg2p, deg2p, hp, bdr, b4)

  dout = Wfin.shape[1]
  bdf = jnp.kron(jnp.eye(4, dtype=jnp.float32), Wfin)
  bf4 = jnp.tile(bfin, 4).reshape(1, 4 * dout)

  def body2(a_ref, d_ref, h_ref, w_ref, b_ref, wf_ref, bf_ref, o_ref):
    agg = a_ref[0] + a_ref[1]
    rdeg = 1.0 / jnp.maximum(d_ref[0] + d_ref[1], 1.0)
    h2 = jnp.maximum(
        agg * rdeg
        + jnp.dot(h_ref[...], w_ref[...], preferred_element_type=jnp.float32)
        + b_ref[...], 0.0)
    o_ref[...] = (
        jnp.dot(h2, wf_ref[...], preferred_element_type=jnp.float32)
        + bf_ref[...])

  return pl.pallas_call(
      body2, out_shape=jax.ShapeDtypeStruct((rows, 4 * dout), jnp.float32))(
          agg2p, deg2p, hp, bdr, b4, bdf, bf4)


def kernel(x, edge_index, edge_attr, W_emb, b_emb, K1, Kb1, K2, Kb2, K3, Kb3,
           W_root, bias, W_inv, b_inv):
  n, _ = x.shape
  e, de = edge_attr.shape
  d = W_emb.shape[1]

  ep = ((e + 4 * LANES - 1) // (4 * LANES)) * (4 * LANES)
  pad = ep - e
  src = jnp.concatenate([edge_index[0], jnp.zeros((pad,), jnp.int32)])
  dst = jnp.concatenate(
      [edge_index[1], jnp.full((pad,), n, jnp.int32)])
  ea_pad = jnp.concatenate([edge_attr, jnp.zeros((pad, de), jnp.float32)])
  eaP8 = ea_pad.reshape(ep // 8, 8 * de)

  chunk = 1024
  gather = _make_gather(ep, d, chunk)
  scatter = _make_scatter_add(n, ep, d, chunk)
  degree = _make_degree(n, ep, chunk)

  zeros_d = jnp.zeros((n + 16, d), jnp.float32)
  zeros_32 = jnp.zeros((n + 16, 32), jnp.float32)
  ones_32 = jnp.ones((chunk, 32), jnp.float32)

  hp = _embed_packed(x, W_emb, b_emb)
  deg2 = degree(dst, ones_32, zeros_32)
  deg2p = deg2.reshape(NC, n // 4, 4 * d)

  for layer in range(2):
    xj = gather(hp.reshape(n, d), src)
    msgP = _edge_messages(
        eaP8, xj.reshape(ep // 4, 128), K1, Kb1, K2, Kb2, K3, Kb3, d)
    agg2 = scatter(msgP.reshape(ep, d), dst, zeros_d)
    agg2p = agg2.reshape(NC, n // 4, 4 * d)
    if layer == 0:
      hp = _update_packed(agg2p, deg2p, hp, W_root, bias)
    else:
      outp = _update_packed(agg2p, deg2p, hp, W_root, bias, W_inv, b_inv)

  return outp.reshape(n, W_inv.shape[1])


# speedup vs baseline: 1.0952x; 1.0952x over previous
import functools

import jax
import jax.numpy as jnp
from jax import lax
from jax.experimental import pallas as pl
from jax.experimental.pallas import tpu as pltpu
from jax.experimental.pallas import tpu_sc as plsc

NC = 2
NS = 16
NW = NC * NS

LANES = 1024


def _sc_mesh():
  return plsc.VectorSubcoreMesh(
      core_axis_name="c", subcore_axis_name="s", num_cores=NC, num_subcores=NS)


_SC_PARAMS = pltpu.CompilerParams(use_tc_tiling_on_sc=False)


def _make_gather(n_rows, d, chunk):
  per_w = n_rows // NW
  assert per_w % chunk == 0
  n_ch = per_w // chunk

  @functools.partial(
      pl.kernel,
      mesh=_sc_mesh(),
      compiler_params=_SC_PARAMS,
      out_type=jax.ShapeDtypeStruct((n_rows, d), jnp.float32),
      scratch_types=[
          pltpu.VMEM((chunk,), jnp.int32),
          pltpu.VMEM((chunk, d), jnp.float32),
          pltpu.SemaphoreType.DMA,
      ],
  )
  def gather_kernel(table_hbm, idx_hbm, out_hbm, idx_v, rows_v, sem):
    wid = lax.axis_index("s") * NC + lax.axis_index("c")
    base = wid * per_w
    for j in range(n_ch):
      off = base + j * chunk
      pltpu.sync_copy(idx_hbm.at[pl.ds(off, chunk)], idx_v)
      pltpu.async_copy(table_hbm.at[idx_v], rows_v, sem).wait()
      pltpu.sync_copy(rows_v, out_hbm.at[pl.ds(off, chunk)])

  return gather_kernel


def _make_scatter_add(n_nodes, n_rows, d, chunk):
  per_w = n_rows // NW
  assert per_w % chunk == 0
  n_ch = per_w // chunk

  @functools.partial(
      pl.kernel,
      mesh=_sc_mesh(),
      compiler_params=_SC_PARAMS,
      out_type=jax.ShapeDtypeStruct((NC, n_nodes, d), jnp.float32),
      scratch_types=[
          pltpu.VMEM((chunk,), jnp.int32),
          pltpu.VMEM((chunk, d), jnp.float32),
          pltpu.VMEM_SHARED((n_nodes + 16, d), jnp.float32),
      ],
  )
  def scatter_kernel(rows_hbm, idx_hbm, zeros_hbm, out_hbm, idx_v, rows_v,
                     acc_sh):
    cid = lax.axis_index("c")
    sid = lax.axis_index("s")

    @pl.when(sid == 0)
    def _init():
      pltpu.sync_copy(zeros_hbm, acc_sh)

    plsc.subcore_barrier()
    wid = sid * NC + cid
    base = wid * per_w
    for j in range(n_ch):
      off = base + j * chunk
      pltpu.sync_copy(idx_hbm.at[pl.ds(off, chunk)], idx_v)
      pltpu.sync_copy(rows_hbm.at[pl.ds(off, chunk)], rows_v)
      pltpu.sync_copy(rows_v, acc_sh.at[idx_v], add=True)
    plsc.subcore_barrier()

    @pl.when(sid == 0)
    def _flush():
      pltpu.sync_copy(acc_sh.at[pl.ds(0, n_nodes)], out_hbm.at[cid])

  return scatter_kernel


def _make_degree(n_nodes, n_rows, chunk):
  per_w = n_rows // NW
  n_ch = per_w // chunk

  @functools.partial(
      pl.kernel,
      mesh=_sc_mesh(),
      compiler_params=_SC_PARAMS,
      out_type=jax.ShapeDtypeStruct((NC, n_nodes, 32), jnp.float32),
      scratch_types=[
          pltpu.VMEM((chunk,), jnp.int32),
          pltpu.VMEM((chunk, 32), jnp.float32),
          pltpu.VMEM_SHARED((n_nodes + 16, 32), jnp.float32),
      ],
  )
  def degree_kernel(idx_hbm, ones_hbm, zeros_hbm, out_hbm, idx_v, ones_v,
                    acc_sh):
    cid = lax.axis_index("c")
    sid = lax.axis_index("s")

    @pl.when(sid == 0)
    def _init():
      pltpu.sync_copy(zeros_hbm, acc_sh)

    pltpu.sync_copy(ones_hbm, ones_v)
    plsc.subcore_barrier()
    wid = sid * NC + cid
    base = wid * per_w
    for j in range(n_ch):
      off = base + j * chunk
      pltpu.sync_copy(idx_hbm.at[pl.ds(off, chunk)], idx_v)
      pltpu.sync_copy(ones_v, acc_sh.at[idx_v], add=True)
    plsc.subcore_barrier()

    @pl.when(sid == 0)
    def _flush():
      pltpu.sync_copy(acc_sh.at[pl.ds(0, n_nodes)], out_hbm.at[cid])

  return degree_kernel


def _embed_packed(x, W, b):
  n, din = x.shape
  d = W.shape[1]
  x4 = x.reshape(n // 4, 4 * din)
  bd = jnp.kron(jnp.eye(4, dtype=jnp.float32), W)
  b4 = jnp.tile(b, 4).reshape(1, 4 * d)

  def body(x_ref, w_ref, b_ref, o_ref):
    o_ref[...] = (
        jnp.dot(x_ref[...], w_ref[...], preferred_element_type=jnp.float32)
        + b_ref[...])

  return pl.pallas_call(
      body, out_shape=jax.ShapeDtypeStruct((n // 4, 4 * d), jnp.float32))(
          x4, bd, b4)


def _edge_messages(eaQ, xjP, K1, Kb1, K2, Kb2, K3, Kb3, d):
  _, de, rows = eaQ.shape
  kw = K1.shape[1]
  assert rows % LANES == 0

  k1t = K1.T
  k2t = K2.T
  k3t = K3.T.astype(jnp.bfloat16)
  kb3t = Kb3.reshape(d, d).T
  kb1c = Kb1.reshape(kw, 1)
  kb2c = Kb2.reshape(kw, 1)

  def body(eaQ_ref, xj_ref, k1_ref, kb1_ref, k2_ref, kb2_ref, k3_ref,
           kb3_ref, o_ref):
    xjT = jnp.transpose(xj_ref[...])
    accs = []
    for k in range(4):
      a = jnp.dot(k1_ref[...], eaQ_ref[k], preferred_element_type=jnp.float32)
      a = jnp.maximum(a + kb1_ref[...], 0.0)
      a = jnp.dot(k2_ref[...], a, preferred_element_type=jnp.float32)
      a = jnp.maximum(a + kb2_ref[...], 0.0)
      wT = jnp.dot(
          k3_ref[...], a.astype(jnp.bfloat16),
          preferred_element_type=jnp.float32).astype(jnp.bfloat16)
      acc = jnp.dot(
          kb3_ref[...], xjT[k * d:(k + 1) * d, :],
          preferred_element_type=jnp.float32)
      for i in range(d):
        acc = acc + (xjT[k * d + i:k * d + i + 1, :] *
                     wT[i * d:(i + 1) * d, :].astype(jnp.float32))
      accs.append(acc)
    o_ref[...] = jnp.transpose(jnp.concatenate(accs, axis=0))

  full = lambda shape: pl.BlockSpec(shape, lambda i: tuple(0 for _ in shape))
  return pl.pallas_call(
      body,
      grid=(rows // LANES,),
      in_specs=[
          pl.BlockSpec((4, de, LANES), lambda i: (0, 0, i)),
          pl.BlockSpec((LANES, 128), lambda i: (i, 0)),
          full((kw, de)),
          full((kw, 1)),
          full((kw, kw)),
          full((kw, 1)),
          full((d * d, kw)),
          full((d, d)),
      ],
      out_specs=pl.BlockSpec((LANES, 128), lambda i: (i, 0)),
      out_shape=jax.ShapeDtypeStruct((rows, 128), jnp.float32),
  )(eaQ, xjP, k1t, kb1c, k2t, kb2c, k3t, kb3t)


def _update_packed(agg2p, deg2p, hp, Wr, b, Wfin=None, bfin=None):
  rows, dl = hp.shape
  bdr = jnp.kron(jnp.eye(4, dtype=jnp.float32), Wr)
  b4 = jnp.tile(b, 4).reshape(1, dl)

  if Wfin is None:
    def body(a_ref, d_ref, h_ref, w_ref, b_ref, o_ref):
      agg = a_ref[0] + a_ref[1]
      rdeg = 1.0 / jnp.maximum(d_ref[0] + d_ref[1], 1.0)
      o_ref[...] = jnp.maximum(
          agg * rdeg
          + jnp.dot(h_ref[...], w_ref[...], preferred_element_type=jnp.float32)
          + b_ref[...], 0.0)

    return pl.pallas_call(
        body, out_shape=jax.ShapeDtypeStruct((rows, dl), jnp.float32))(
            agg2p, deg2p, hp, bdr, b4)

  dout = Wfin.shape[1]
  bdf = jnp.kron(jnp.eye(4, dtype=jnp.float32), Wfin)
  bf4 = jnp.tile(bfin, 4).reshape(1, 4 * dout)

  def body2(a_ref, d_ref, h_ref, w_ref, b_ref, wf_ref, bf_ref, o_ref):
    agg = a_ref[0] + a_ref[1]
    rdeg = 1.0 / jnp.maximum(d_ref[0] + d_ref[1], 1.0)
    h2 = jnp.maximum(
        agg * rdeg
        + jnp.dot(h_ref[...], w_ref[...], preferred_element_type=jnp.float32)
        + b_ref[...], 0.0)
    o_ref[...] = (
        jnp.dot(h2, wf_ref[...], preferred_element_type=jnp.float32)
        + bf_ref[...])

  return pl.pallas_call(
      body2, out_shape=jax.ShapeDtypeStruct((rows, 4 * dout), jnp.float32))(
          agg2p, deg2p, hp, bdr, b4, bdf, bf4)


def kernel(x, edge_index, edge_attr, W_emb, b_emb, K1, Kb1, K2, Kb2, K3, Kb3,
           W_root, bias, W_inv, b_inv):
  n, _ = x.shape
  e, de = edge_attr.shape
  d = W_emb.shape[1]

  ep = ((e + 4 * LANES - 1) // (4 * LANES)) * (4 * LANES)
  pad = ep - e
  src = jnp.concatenate([edge_index[0], jnp.zeros((pad,), jnp.int32)])
  dst = jnp.concatenate(
      [edge_index[1], jnp.full((pad,), n, jnp.int32)])
  ea_pad = jnp.concatenate([edge_attr, jnp.zeros((pad, de), jnp.float32)])
  eaQ = ea_pad.reshape(ep // 4, 4, de).transpose(1, 2, 0)

  chunk = 2560
  gather = _make_gather(ep, d, chunk)
  scatter = _make_scatter_add(n, ep, d, chunk)
  degree = _make_degree(n, ep, chunk)

  zeros_d = jnp.zeros((n + 16, d), jnp.float32)
  zeros_32 = jnp.zeros((n + 16, 32), jnp.float32)
  ones_32 = jnp.ones((chunk, 32), jnp.float32)

  hp = _embed_packed(x, W_emb, b_emb)
  deg2 = degree(dst, ones_32, zeros_32)
  deg2p = deg2.reshape(NC, n // 4, 4 * d)

  for layer in range(2):
    xj = gather(hp.reshape(n, d), src)
    msgP = _edge_messages(
        eaQ, xj.reshape(ep // 4, 128), K1, Kb1, K2, Kb2, K3, Kb3, d)
    agg2 = scatter(msgP.reshape(ep, d), dst, zeros_d)
    agg2p = agg2.reshape(NC, n // 4, 4 * d)
    if layer == 0:
      hp = _update_packed(agg2p, deg2p, hp, W_root, bias)
    else:
      outp = _update_packed(agg2p, deg2p, hp, W_root, bias, W_inv, b_inv)

  return outp.reshape(n, W_inv.shape[1])


# speedup vs baseline: 1.1016x; 1.0058x over previous
import functools

import jax
import jax.numpy as jnp
from jax import lax
from jax.experimental import pallas as pl
from jax.experimental.pallas import tpu as pltpu
from jax.experimental.pallas import tpu_sc as plsc

NC = 2
NS = 16
NW = NC * NS

LANES = 1024


def _sc_mesh():
  return plsc.VectorSubcoreMesh(
      core_axis_name="c", subcore_axis_name="s", num_cores=NC, num_subcores=NS)


_SC_PARAMS = pltpu.CompilerParams(use_tc_tiling_on_sc=False)


def _make_gather(n_rows, d, chunk):
  per_w = n_rows // NW
  assert per_w % chunk == 0
  n_ch = per_w // chunk

  @functools.partial(
      pl.kernel,
      mesh=_sc_mesh(),
      compiler_params=_SC_PARAMS,
      out_type=jax.ShapeDtypeStruct((n_rows, d), jnp.float32),
      scratch_types=[
          pltpu.VMEM((chunk,), jnp.int32),
          pltpu.VMEM((chunk,), jnp.int32),
          pltpu.VMEM((chunk, d), jnp.float32),
          pltpu.VMEM((chunk, d), jnp.float32),
          pltpu.SemaphoreType.DMA,
          pltpu.SemaphoreType.DMA,
          pltpu.SemaphoreType.DMA,
          pltpu.SemaphoreType.DMA,
      ],
  )
  def gather_kernel(table_hbm, idx_hbm, out_hbm, idx0, idx1, rows0, rows1,
                    si0, si1, sw0, sw1):
    wid = lax.axis_index("s") * NC + lax.axis_index("c")
    base = wid * per_w
    idx = [idx0, idx1]
    rows = [rows0, rows1]
    si = [si0, si1]
    sw = [sw0, sw1]
    pltpu.async_copy(idx_hbm.at[pl.ds(base, chunk)], idx[0], si[0])
    for j in range(n_ch):
      b = j % 2
      off = base + j * chunk
      if j + 1 < n_ch:
        pltpu.async_copy(
            idx_hbm.at[pl.ds(off + chunk, chunk)], idx[1 - b], si[1 - b])
      pltpu.make_async_copy(
          idx_hbm.at[pl.ds(off, chunk)], idx[b], si[b]).wait()
      if j >= 2:
        pltpu.make_async_copy(
            rows[b], out_hbm.at[pl.ds(off - 2 * chunk, chunk)], sw[b]).wait()
      pltpu.async_copy(table_hbm.at[idx[b]], rows[b], si[b]).wait()
      pltpu.async_copy(rows[b], out_hbm.at[pl.ds(off, chunk)], sw[b])
    for j in range(max(n_ch - 2, 0), n_ch):
      b = j % 2
      pltpu.make_async_copy(
          rows[b], out_hbm.at[pl.ds(base + j * chunk, chunk)], sw[b]).wait()

  return gather_kernel


def _make_scatter_add(n_nodes, n_rows, d, chunk):
  per_w = n_rows // NW
  assert per_w % chunk == 0
  n_ch = per_w // chunk

  @functools.partial(
      pl.kernel,
      mesh=_sc_mesh(),
      compiler_params=_SC_PARAMS,
      out_type=jax.ShapeDtypeStruct((NC, n_nodes, d), jnp.float32),
      scratch_types=[
          pltpu.VMEM((chunk,), jnp.int32),
          pltpu.VMEM((chunk, d), jnp.float32),
          pltpu.VMEM_SHARED((n_nodes + 16, d), jnp.float32),
      ],
  )
  def scatter_kernel(rows_hbm, idx_hbm, zeros_hbm, out_hbm, idx_v, rows_v,
                     acc_sh):
    cid = lax.axis_index("c")
    sid = lax.axis_index("s")

    @pl.when(sid == 0)
    def _init():
      pltpu.sync_copy(zeros_hbm, acc_sh)

    plsc.subcore_barrier()
    wid = sid * NC + cid
    base = wid * per_w
    for j in range(n_ch):
      off = base + j * chunk
      pltpu.sync_copy(idx_hbm.at[pl.ds(off, chunk)], idx_v)
      pltpu.sync_copy(rows_hbm.at[pl.ds(off, chunk)], rows_v)
      pltpu.sync_copy(rows_v, acc_sh.at[idx_v], add=True)
    plsc.subcore_barrier()

    @pl.when(sid == 0)
    def _flush():
      pltpu.sync_copy(acc_sh.at[pl.ds(0, n_nodes)], out_hbm.at[cid])

  return scatter_kernel


def _make_degree(n_nodes, n_rows, chunk):
  per_w = n_rows // NW
  n_ch = per_w // chunk

  @functools.partial(
      pl.kernel,
      mesh=_sc_mesh(),
      compiler_params=_SC_PARAMS,
      out_type=jax.ShapeDtypeStruct((NC, n_nodes, 32), jnp.float32),
      scratch_types=[
          pltpu.VMEM((chunk,), jnp.int32),
          pltpu.VMEM((chunk, 32), jnp.float32),
          pltpu.VMEM_SHARED((n_nodes + 16, 32), jnp.float32),
      ],
  )
  def degree_kernel(idx_hbm, ones_hbm, zeros_hbm, out_hbm, idx_v, ones_v,
                    acc_sh):
    cid = lax.axis_index("c")
    sid = lax.axis_index("s")

    @pl.when(sid == 0)
    def _init():
      pltpu.sync_copy(zeros_hbm, acc_sh)

    pltpu.sync_copy(ones_hbm, ones_v)
    plsc.subcore_barrier()
    wid = sid * NC + cid
    base = wid * per_w
    for j in range(n_ch):
      off = base + j * chunk
      pltpu.sync_copy(idx_hbm.at[pl.ds(off, chunk)], idx_v)
      pltpu.sync_copy(ones_v, acc_sh.at[idx_v], add=True)
    plsc.subcore_barrier()

    @pl.when(sid == 0)
    def _flush():
      pltpu.sync_copy(acc_sh.at[pl.ds(0, n_nodes)], out_hbm.at[cid])

  return degree_kernel


def _embed_packed(x, W, b):
  n, din = x.shape
  d = W.shape[1]
  x4 = x.reshape(n // 4, 4 * din)
  bd = jnp.kron(jnp.eye(4, dtype=jnp.float32), W)
  b4 = jnp.tile(b, 4).reshape(1, 4 * d)

  def body(x_ref, w_ref, b_ref, o_ref):
    o_ref[...] = (
        jnp.dot(x_ref[...], w_ref[...], preferred_element_type=jnp.float32)
        + b_ref[...])

  return pl.pallas_call(
      body, out_shape=jax.ShapeDtypeStruct((n // 4, 4 * d), jnp.float32))(
          x4, bd, b4)


def _edge_messages(eaQ, xjP, K1, Kb1, K2, Kb2, K3, Kb3, d):
  _, de, rows = eaQ.shape
  kw = K1.shape[1]
  assert rows % LANES == 0

  k1t = K1.T
  k2t = K2.T
  k3t = K3.T.astype(jnp.bfloat16)
  kb3t = Kb3.reshape(d, d).T
  kb1c = Kb1.reshape(kw, 1)
  kb2c = Kb2.reshape(kw, 1)

  def body(eaQ_ref, xj_ref, k1_ref, kb1_ref, k2_ref, kb2_ref, k3_ref,
           kb3_ref, o_ref):
    xjT = jnp.transpose(xj_ref[...])
    accs = []
    for k in range(4):
      a = jnp.dot(k1_ref[...], eaQ_ref[k], preferred_element_type=jnp.float32)
      a = jnp.maximum(a + kb1_ref[...], 0.0)
      a = jnp.dot(k2_ref[...], a, preferred_element_type=jnp.float32)
      a = jnp.maximum(a + kb2_ref[...], 0.0)
      wT = jnp.dot(
          k3_ref[...], a.astype(jnp.bfloat16),
          preferred_element_type=jnp.float32).astype(jnp.bfloat16)
      acc = jnp.dot(
          kb3_ref[...], xjT[k * d:(k + 1) * d, :],
          preferred_element_type=jnp.float32)
      for i in range(d):
        acc = acc + (xjT[k * d + i:k * d + i + 1, :] *
                     wT[i * d:(i + 1) * d, :].astype(jnp.float32))
      accs.append(acc)
    o_ref[...] = jnp.transpose(jnp.concatenate(accs, axis=0))

  full = lambda shape: pl.BlockSpec(shape, lambda i: tuple(0 for _ in shape))
  return pl.pallas_call(
      body,
      grid=(rows // LANES,),
      in_specs=[
          pl.BlockSpec((4, de, LANES), lambda i: (0, 0, i)),
          pl.BlockSpec((LANES, 128), lambda i: (i, 0)),
          full((kw, de)),
          full((kw, 1)),
          full((kw, kw)),
          full((kw, 1)),
          full((d * d, kw)),
          full((d, d)),
      ],
      out_specs=pl.BlockSpec((LANES, 128), lambda i: (i, 0)),
      out_shape=jax.ShapeDtypeStruct((rows, 128), jnp.float32),
  )(eaQ, xjP, k1t, kb1c, k2t, kb2c, k3t, kb3t)


def _update_packed(agg2p, deg2p, hp, Wr, b, Wfin=None, bfin=None):
  rows, dl = hp.shape
  bdr = jnp.kron(jnp.eye(4, dtype=jnp.float32), Wr)
  b4 = jnp.tile(b, 4).reshape(1, dl)

  if Wfin is None:
    def body(a_ref, d_ref, h_ref, w_ref, b_ref, o_ref):
      agg = a_ref[0] + a_ref[1]
      rdeg = 1.0 / jnp.maximum(d_ref[0] + d_ref[1], 1.0)
      o_ref[...] = jnp.maximum(
          agg * rdeg
          + jnp.dot(h_ref[...], w_ref[...], preferred_element_type=jnp.float32)
          + b_ref[...], 0.0)

    return pl.pallas_call(
        body, out_shape=jax.ShapeDtypeStruct((rows, dl), jnp.float32))(
            agg2p, deg2p, hp, bdr, b4)

  dout = Wfin.shape[1]
  bdf = jnp.kron(jnp.eye(4, dtype=jnp.float32), Wfin)
  bf4 = jnp.tile(bfin, 4).reshape(1, 4 * dout)

  def body2(a_ref, d_ref, h_ref, w_ref, b_ref, wf_ref, bf_ref, o_ref):
    agg = a_ref[0] + a_ref[1]
    rdeg = 1.0 / jnp.maximum(d_ref[0] + d_ref[1], 1.0)
    h2 = jnp.maximum(
        agg * rdeg
        + jnp.dot(h_ref[...], w_ref[...], preferred_element_type=jnp.float32)
        + b_ref[...], 0.0)
    o_ref[...] = (
        jnp.dot(h2, wf_ref[...], preferred_element_type=jnp.float32)
        + bf_ref[...])

  return pl.pallas_call(
      body2, out_shape=jax.ShapeDtypeStruct((rows, 4 * dout), jnp.float32))(
          agg2p, deg2p, hp, bdr, b4, bdf, bf4)


def kernel(x, edge_index, edge_attr, W_emb, b_emb, K1, Kb1, K2, Kb2, K3, Kb3,
           W_root, bias, W_inv, b_inv):
  n, _ = x.shape
  e, de = edge_attr.shape
  d = W_emb.shape[1]

  ep = ((e + 4 * LANES - 1) // (4 * LANES)) * (4 * LANES)
  pad = ep - e
  src = jnp.concatenate([edge_index[0], jnp.zeros((pad,), jnp.int32)])
  dst = jnp.concatenate(
      [edge_index[1], jnp.full((pad,), n, jnp.int32)])
  ea_pad = jnp.concatenate([edge_attr, jnp.zeros((pad, de), jnp.float32)])
  eaQ = ea_pad.reshape(ep // 4, 4, de).transpose(1, 2, 0)

  chunk = 2560
  gather = _make_gather(ep, d, 1024)
  scatter = _make_scatter_add(n, ep, d, chunk)
  degree = _make_degree(n, ep, chunk)

  zeros_d = jnp.zeros((n + 16, d), jnp.float32)
  zeros_32 = jnp.zeros((n + 16, 32), jnp.float32)
  ones_32 = jnp.ones((chunk, 32), jnp.float32)

  hp = _embed_packed(x, W_emb, b_emb)
  deg2 = degree(dst, ones_32, zeros_32)
  deg2p = deg2.reshape(NC, n // 4, 4 * d)

  for layer in range(2):
    xj = gather(hp.reshape(n, d), src)
    msgP = _edge_messages(
        eaQ, xj.reshape(ep // 4, 128), K1, Kb1, K2, Kb2, K3, Kb3, d)
    agg2 = scatter(msgP.reshape(ep, d), dst, zeros_d)
    agg2p = agg2.reshape(NC, n // 4, 4 * d)
    if layer == 0:
      hp = _update_packed(agg2p, deg2p, hp, W_root, bias)
    else:
      outp = _update_packed(agg2p, deg2p, hp, W_root, bias, W_inv, b_inv)

  return outp.reshape(n, W_inv.shape[1])
